# baseline (speedup 1.0000x reference)
import jax
import jax.numpy as jnp
from jax import lax
from jax.experimental import pallas as pl
from jax.experimental.pallas import tpu as pltpu
from jax.experimental.pallas import tpu_sc as plsc

BATCH = 4096
FIELDS = 26
EMBED_DIM = 16
NUM_CORES = 2
NUM_SUBCORES = 16
NUM_TILES = NUM_CORES * NUM_SUBCORES
B_PER_TILE = BATCH // NUM_TILES
EMB_ROWS = FIELDS * EMBED_DIM
F_SPLIT = 20


def _fm_body(sparse_hbm, embed_hbm, w_hbm, out_hbm,
             idx_v, gath_v, emb_v, s_v, out_v, sem_e, sem_g):
    wid = lax.axis_index("s") * NUM_CORES + lax.axis_index("c")
    b0 = wid * B_PER_TILE

    pltpu.sync_copy(sparse_hbm.at[:, pl.ds(b0, B_PER_TILE)], idx_v)
    cp_e0 = pltpu.async_copy(
        embed_hbm.at[pl.ds(0, F_SPLIT * EMBED_DIM), pl.ds(b0, B_PER_TILE)],
        emb_v.at[pl.ds(0, F_SPLIT * EMBED_DIM)], sem_e)
    cp_e1 = pltpu.async_copy(
        embed_hbm.at[pl.ds(F_SPLIT * EMBED_DIM, EMB_ROWS - F_SPLIT * EMBED_DIM),
                     pl.ds(b0, B_PER_TILE)],
        emb_v.at[pl.ds(F_SPLIT * EMBED_DIM, EMB_ROWS - F_SPLIT * EMBED_DIM)],
        sem_e)
    gather_cps = []
    for f in range(FIELDS):
        gather_cps.append(
            pltpu.async_copy(w_hbm.at[0].at[idx_v.at[f]], gath_v.at[f],
                             sem_g))

    cp_e0.wait()

    def c0_body(c, _):
        col = c * 16
        ssq = emb_v[0, pl.ds(col, 16)] * 0.0
        for d in range(EMBED_DIM):
            v = emb_v[d, pl.ds(col, 16)]
            s = v
            ssq = ssq + v * v
            for f in range(1, F_SPLIT):
                v = emb_v[f * EMBED_DIM + d, pl.ds(col, 16)]
                s = s + v
                ssq = ssq + v * v
            s_v[pl.ds((c * EMBED_DIM + d) * 16, 16)] = s
        out_v[0, pl.ds(col, 16)] = ssq
        return 0

    lax.fori_loop(0, B_PER_TILE // 16, c0_body, 0)

    cp_e1.wait()

    def c1_body(c, _):
        col = c * 16
        ssq = out_v[0, pl.ds(col, 16)]
        sos = ssq * 0.0
        for d in range(EMBED_DIM):
            s = s_v[pl.ds((c * EMBED_DIM + d) * 16, 16)]
            for f in range(F_SPLIT, FIELDS):
                v = emb_v[f * EMBED_DIM + d, pl.ds(col, 16)]
                s = s + v
                ssq = ssq + v * v
            sos = sos + s * s
        out_v[0, pl.ds(col, 16)] = 0.5 * (sos - ssq)
        return 0

    lax.fori_loop(0, B_PER_TILE // 16, c1_body, 0)

    for cp in gather_cps:
        cp.wait()

    def a_body(c, _):
        col = c * 16
        first = gath_v[0, pl.ds(col, 16)]
        for f in range(1, FIELDS):
            first = first + gath_v[f, pl.ds(col, 16)]
        out_v[0, pl.ds(col, 16)] = out_v[0, pl.ds(col, 16)] + first
        return 0

    lax.fori_loop(0, B_PER_TILE // 16, a_body, 0)

    pltpu.sync_copy(out_v, out_hbm.at[wid])


@jax.jit
def _fm_kernel(sparse_t, embed_t, w_flat):
    run = pl.kernel(
        _fm_body,
        out_type=jax.ShapeDtypeStruct((NUM_TILES, 1, B_PER_TILE), jnp.float32),
        mesh=plsc.VectorSubcoreMesh(core_axis_name="c", subcore_axis_name="s",
                                    num_cores=NUM_CORES,
                                    num_subcores=NUM_SUBCORES),
        scratch_types=[
            pltpu.VMEM((FIELDS, B_PER_TILE), jnp.int32),
            pltpu.VMEM((FIELDS, B_PER_TILE), jnp.float32),
            pltpu.VMEM((EMB_ROWS, B_PER_TILE), jnp.float32),
            pltpu.VMEM((B_PER_TILE * EMBED_DIM,), jnp.float32),
            pltpu.VMEM((1, B_PER_TILE), jnp.float32),
            pltpu.SemaphoreType.DMA,
            pltpu.SemaphoreType.DMA,
        ],
        compiler_params=pltpu.CompilerParams(needs_layout_passes=False),
    )
    return run(sparse_t, embed_t, w_flat)


def kernel(sparse_inputs, embed_inputs, w):
    sparse_t = sparse_inputs.T
    embed_t = jnp.transpose(embed_inputs, (1, 2, 0)).reshape(EMB_ROWS, BATCH)
    w_t = w.T
    out = _fm_kernel(sparse_t, embed_t, w_t)
    return out.reshape(BATCH, 1)


# speedup vs baseline: 1.1881x; 1.1881x over previous
import functools

import jax
import jax.numpy as jnp
from jax import lax
from jax.experimental import pallas as pl
from jax.experimental.pallas import tpu as pltpu
from jax.experimental.pallas import tpu_sc as plsc

BATCH = 4096
FIELDS = 26
EMBED_DIM = 16
NUM_CORES = 2
NUM_SUBCORES = 16
NUM_TILES = NUM_CORES * NUM_SUBCORES
B_PER_TILE = BATCH // NUM_TILES
EMB_ROWS = FIELDS * EMBED_DIM
TC_BLOCK = 512


def _first_order_body(sparse_hbm, w_hbm, out_hbm, idx_v, gath_v, out_v,
                      sem_g):
    wid = lax.axis_index("s") * NUM_CORES + lax.axis_index("c")
    b0 = wid * B_PER_TILE

    pltpu.sync_copy(sparse_hbm.at[:, pl.ds(b0, B_PER_TILE)], idx_v)
    gather_cps = []
    for f in range(FIELDS):
        gather_cps.append(
            pltpu.async_copy(w_hbm.at[0].at[idx_v.at[f]], gath_v.at[f],
                             sem_g))
    for cp in gather_cps:
        cp.wait()

    def a_body(c, _):
        col = c * 16
        first = gath_v[0, pl.ds(col, 16)]
        for f in range(1, FIELDS):
            first = first + gath_v[f, pl.ds(col, 16)]
        out_v[0, pl.ds(col, 16)] = first
        return 0

    lax.fori_loop(0, B_PER_TILE // 16, a_body, 0)

    pltpu.sync_copy(out_v, out_hbm.at[wid])


def _first_order(sparse_t, w_t):
    run = pl.kernel(
        _first_order_body,
        out_type=jax.ShapeDtypeStruct((NUM_TILES, 1, B_PER_TILE), jnp.float32),
        mesh=plsc.VectorSubcoreMesh(core_axis_name="c", subcore_axis_name="s",
                                    num_cores=NUM_CORES,
                                    num_subcores=NUM_SUBCORES),
        scratch_types=[
            pltpu.VMEM((FIELDS, B_PER_TILE), jnp.int32),
            pltpu.VMEM((FIELDS, B_PER_TILE), jnp.float32),
            pltpu.VMEM((1, B_PER_TILE), jnp.float32),
            pltpu.SemaphoreType.DMA,
        ],
        compiler_params=pltpu.CompilerParams(needs_layout_passes=False),
    )
    return run(sparse_t, w_t)


def _second_order_body(e_ref, o_ref):
    e = e_ref[...]
    r = e.reshape(FIELDS, EMBED_DIM, TC_BLOCK)
    s = jnp.sum(r, axis=0)
    ssq = jnp.sum(e * e, axis=0, keepdims=True)
    sos = jnp.sum(s * s, axis=0, keepdims=True)
    o_ref[...] = 0.5 * (sos - ssq)


def _second_order(embed_t):
    return pl.pallas_call(
        _second_order_body,
        grid=(BATCH // TC_BLOCK,),
        in_specs=[pl.BlockSpec((EMB_ROWS, TC_BLOCK), lambda i: (0, i))],
        out_specs=pl.BlockSpec((1, TC_BLOCK), lambda i: (0, i)),
        out_shape=jax.ShapeDtypeStruct((1, BATCH), jnp.float32),
    )(embed_t)


@jax.jit
def kernel(sparse_inputs, embed_inputs, w):
    sparse_t = sparse_inputs.T
    embed_t = jnp.transpose(embed_inputs, (1, 2, 0)).reshape(EMB_ROWS, BATCH)
    w_t = w.T
    first = _first_order(sparse_t, w_t).reshape(BATCH)
    second = _second_order(embed_t).reshape(BATCH)
    return (first + second).reshape(BATCH, 1)
